# baseline (speedup 1.0000x reference)
import functools

import jax
import jax.numpy as jnp
from jax import lax
from jax.experimental import pallas as pl
from jax.experimental.pallas import tpu as pltpu
from jax.experimental.pallas import tpu_sc as plsc

_CHUNK = 128
_TW = 2048


def _tc_transpose_pair(a_t, b_t):
  _, N = a_t.shape
  G = pl.cdiv(N, _TW)
  NP = G * _TW

  def body(a_r, b_r, oa_r, ob_r):
    oa_r[...] = a_r[...].T
    ob_r[...] = b_r[...].T

  col = pl.BlockSpec((64, _TW), lambda i: (0, i))
  row = pl.BlockSpec((_TW, 64), lambda i: (i, 0))
  return pl.pallas_call(
      body,
      grid=(G,),
      in_specs=[col, col],
      out_specs=[row, row],
      out_shape=[jax.ShapeDtypeStruct((NP, 64), jnp.float32)] * 2,
  )(a_t, b_t)


def _sc_gather_pair(idx, t_a, t_b):
  B = idx.shape[0]
  D = t_a.shape[1]
  info = plsc.get_sparse_core_info()
  NC, NS = info.num_cores, info.num_subcores
  NW = NC * NS
  assert B % (NW * _CHUNK) == 0
  bpw = B // NW
  ch = bpw // _CHUNK

  idx2 = idx.reshape(NW * ch, _CHUNK)

  mesh = plsc.VectorSubcoreMesh(core_axis_name="c", subcore_axis_name="s")
  out_t = [jax.ShapeDtypeStruct((B, D), jnp.float32) for _ in range(2)]

  @functools.partial(
      pl.kernel,
      out_type=out_t,
      mesh=mesh,
      compiler_params=pltpu.CompilerParams(use_tc_tiling_on_sc=False),
      scratch_types=[
          pltpu.VMEM((ch, _CHUNK), jnp.int32),
          pltpu.VMEM((bpw, D), jnp.float32),
          pltpu.VMEM((bpw, D), jnp.float32),
          pltpu.SemaphoreType.DMA,
          pltpu.SemaphoreType.DMA,
      ],
  )
  def k(ix, ta, tb, o_a, o_b, idx_r, rows0, rows1, sem0, sem1):
    wid = lax.axis_index("s") * NC + lax.axis_index("c")
    base = wid * bpw
    pltpu.sync_copy(ix.at[pl.ds(wid * ch, ch)], idx_r)

    descs_a = [
        pltpu.async_copy(ta.at[idx_r.at[j]],
                         rows0.at[pl.ds(j * _CHUNK, _CHUNK)], sem0)
        for j in range(ch)
    ]
    descs_b = [
        pltpu.async_copy(tb.at[idx_r.at[j]],
                         rows1.at[pl.ds(j * _CHUNK, _CHUNK)], sem1)
        for j in range(ch)
    ]
    for d in descs_a:
      d.wait()
    pltpu.sync_copy(rows0, o_a.at[pl.ds(base, bpw)])
    for d in descs_b:
      d.wait()
    pltpu.sync_copy(rows1, o_b.at[pl.ds(base, bpw)])

  return k(idx2, t_a, t_b)


def _tc_mlp(gu, gi, mu, mi, W1, b1, W2, b2, W3, b3, Wo, bo):
  B2, D2 = gu.shape
  D = D2 // 2
  bb = 1024
  grid = B2 // bb

  def body(gu_r, gi_r, mu_r, mi_r, W1_r, b1_r, W2_r, b2_r, W3_r, b3_r,
           Wo_r, bo_r, oe_r, oo_r):
    dot = functools.partial(jnp.dot, precision=lax.Precision.HIGHEST,
                            preferred_element_type=jnp.float32)
    H3 = W3_r.shape[1]

    def tower(guh, gih, muh, mih, o_r):
      h = dot(muh, W1_r[0:D, :]) + dot(mih, W1_r[D:2 * D, :])
      h = jnp.maximum(h + b1_r[...], 0.0)
      h = jnp.maximum(dot(h, W2_r[...]) + b2_r[...], 0.0)
      h = jnp.maximum(dot(h, W3_r[...]) + b3_r[...], 0.0)
      g = guh * gih
      logit = dot(g, Wo_r[0:D, :]) + dot(h, Wo_r[D:D + H3, :]) + bo_r[...]
      o_r[...] = jax.nn.sigmoid(logit[:, 0])

    gu2, gi2, mu2, mi2 = gu_r[...], gi_r[...], mu_r[...], mi_r[...]
    tower(gu2[:, 0:D], gi2[:, 0:D], mu2[:, 0:D], mi2[:, 0:D], oe_r)
    tower(gu2[:, D:2 * D], gi2[:, D:2 * D], mu2[:, D:2 * D], mi2[:, D:2 * D],
          oo_r)

  row = pl.BlockSpec((bb, D2), lambda i: (i, 0))
  full2 = lambda a: pl.BlockSpec(a.shape, lambda i: (0,) * a.ndim)
  oe, oo = pl.pallas_call(
      body,
      grid=(grid,),
      in_specs=[row, row, row, row,
                full2(W1), full2(b1), full2(W2), full2(b2),
                full2(W3), full2(b3), full2(Wo), full2(bo)],
      out_specs=[pl.BlockSpec((bb,), lambda i: (i,)),
                 pl.BlockSpec((bb,), lambda i: (i,))],
      out_shape=[jax.ShapeDtypeStruct((B2,), jnp.float32),
                 jax.ShapeDtypeStruct((B2,), jnp.float32)],
  )(gu, gi, mu, mi, W1, b1, W2, b2, W3, b3, Wo, bo)
  return oe, oo


def kernel(user_indices, item_indices, user_emb_gmf, item_emb_gmf,
           user_emb_mlp, item_emb_mlp, W1, b1, W2, b2, W3, b3, Wo, bo):
  ug2, um2 = _tc_transpose_pair(user_emb_gmf.T, user_emb_mlp.T)
  gu, mu = _sc_gather_pair(user_indices, ug2, um2)
  ig2, im2 = _tc_transpose_pair(item_emb_gmf.T, item_emb_mlp.T)
  gi, mi = _sc_gather_pair(item_indices, ig2, im2)
  B, D = gu.shape
  pack = lambda a: a.reshape(B // 2, 2 * D)
  oe, oo = _tc_mlp(pack(gu), pack(gi), pack(mu), pack(mi),
                   W1, b1, W2, b2, W3, b3, Wo, bo)
  return jnp.stack((oe, oo), axis=-1).reshape(B)


# speedup vs baseline: 1.9957x; 1.9957x over previous
import functools

import jax
import jax.numpy as jnp
from jax import lax
from jax.experimental import pallas as pl
from jax.experimental.pallas import tpu as pltpu
from jax.experimental.pallas import tpu_sc as plsc

_CHUNK = 128
_TW = 2048
_TH = 25 * _TW


def _tc_pack_transpose4(tables):
  G = _TH // _TW

  def body(*refs):
    ins, outs = refs[:8], refs[8:]
    for t in range(4):
      lo, hi = ins[2 * t][...], ins[2 * t + 1][...]
      outs[t][...] = jnp.concatenate([lo.T, hi.T], axis=1)

  last_blk = (tables[0].shape[1] - 1) // _TW
  lo_spec = pl.BlockSpec((64, _TW), lambda i: (0, i))
  hi_spec = pl.BlockSpec((64, _TW), lambda i: (0, jnp.minimum(i + G, last_blk)))
  out_spec = pl.BlockSpec((_TW, 128), lambda i: (i, 0))
  return pl.pallas_call(
      body,
      grid=(G,),
      in_specs=[s for _ in range(4) for s in (lo_spec, hi_spec)],
      out_specs=[out_spec] * 4,
      out_shape=[jax.ShapeDtypeStruct((_TH, 128), jnp.float32)] * 4,
  )(*[t for tbl in tables for t in (tbl, tbl)])


def _sc_gather4(u_idx, i_idx, t_ug, t_ig, t_um, t_im):
  B = u_idx.shape[0]
  D = t_ug.shape[1]
  info = plsc.get_sparse_core_info()
  NC, NS = info.num_cores, info.num_subcores
  NW = NC * NS
  assert B % (NW * _CHUNK) == 0
  bpw = B // NW
  ch = bpw // _CHUNK

  u_idx2 = u_idx.reshape(NW * ch, _CHUNK)
  i_idx2 = i_idx.reshape(NW * ch, _CHUNK)

  mesh = plsc.VectorSubcoreMesh(core_axis_name="c", subcore_axis_name="s")
  out_t = [jax.ShapeDtypeStruct((B, D), jnp.float32) for _ in range(4)]

  @functools.partial(
      pl.kernel,
      out_type=out_t,
      mesh=mesh,
      compiler_params=pltpu.CompilerParams(use_tc_tiling_on_sc=False),
      scratch_types=[
          pltpu.VMEM((ch, _CHUNK), jnp.int32),
          pltpu.VMEM((ch, _CHUNK), jnp.int32),
          pltpu.VMEM((bpw, D), jnp.float32),
          pltpu.VMEM((bpw, D), jnp.float32),
          pltpu.SemaphoreType.DMA,
          pltpu.SemaphoreType.DMA,
      ],
  )
  def k(uix, iix, ug, ig, um, im, o_ug, o_ig, o_um, o_im,
        idx_u, idx_i, rows0, rows1, sem0, sem1):
    wid = lax.axis_index("s") * NC + lax.axis_index("c")
    base = wid * bpw
    pltpu.sync_copy(uix.at[pl.ds(wid * ch, ch)], idx_u)
    pltpu.sync_copy(iix.at[pl.ds(wid * ch, ch)], idx_i)

    plan = ((ug, idx_u, o_ug, rows0, sem0),
            (ig, idx_i, o_ig, rows1, sem1),
            (um, idx_u, o_um, rows0, sem0),
            (im, idx_i, o_im, rows1, sem1))

    descs = [None, None, None, None]
    for t in range(2):
      tbl, idx, _, rows, sem = plan[t]
      descs[t] = [
          pltpu.async_copy(tbl.at[idx.at[j]],
                           rows.at[pl.ds(j * _CHUNK, _CHUNK)], sem)
          for j in range(ch)
      ]
    for t in range(4):
      _, _, out, rows, _ = plan[t]
      for d in descs[t]:
        d.wait()
      pltpu.sync_copy(rows, out.at[pl.ds(base, bpw)])
      if t + 2 < 4:
        tbl2, idx2, _, rows2, sem2 = plan[t + 2]
        descs[t + 2] = [
            pltpu.async_copy(tbl2.at[idx2.at[j]],
                             rows2.at[pl.ds(j * _CHUNK, _CHUNK)], sem2)
            for j in range(ch)
        ]

  return k(u_idx2, i_idx2, t_ug, t_ig, t_um, t_im)


def _tc_mlp(gu, gi, mu, mi, W1, b1, W2, b2, W3, b3, Wo, bo):
  B2, D2 = gu.shape
  D = D2 // 2
  bb = 1024
  grid = B2 // bb

  def body(gu_r, gi_r, mu_r, mi_r, W1_r, b1_r, W2_r, b2_r, W3_r, b3_r,
           Wo_r, bo_r, oe_r, oo_r):
    dot = functools.partial(jnp.dot, precision=lax.Precision.HIGHEST,
                            preferred_element_type=jnp.float32)
    H3 = W3_r.shape[1]

    def tower(guh, gih, muh, mih, o_r):
      h = dot(muh, W1_r[0:D, :]) + dot(mih, W1_r[D:2 * D, :])
      h = jnp.maximum(h + b1_r[...], 0.0)
      h = jnp.maximum(dot(h, W2_r[...]) + b2_r[...], 0.0)
      h = jnp.maximum(dot(h, W3_r[...]) + b3_r[...], 0.0)
      g = guh * gih
      logit = dot(g, Wo_r[0:D, :]) + dot(h, Wo_r[D:D + H3, :]) + bo_r[...]
      o_r[...] = jax.nn.sigmoid(logit[:, 0])

    gu2, gi2, mu2, mi2 = gu_r[...], gi_r[...], mu_r[...], mi_r[...]
    tower(gu2[:, 0:D], gi2[:, 0:D], mu2[:, 0:D], mi2[:, 0:D], oe_r)
    tower(gu2[:, D:2 * D], gi2[:, D:2 * D], mu2[:, D:2 * D], mi2[:, D:2 * D],
          oo_r)

  row = pl.BlockSpec((bb, D2), lambda i: (i, 0))
  full2 = lambda a: pl.BlockSpec(a.shape, lambda i: (0,) * a.ndim)
  oe, oo = pl.pallas_call(
      body,
      grid=(grid,),
      in_specs=[row, row, row, row,
                full2(W1), full2(b1), full2(W2), full2(b2),
                full2(W3), full2(b3), full2(Wo), full2(bo)],
      out_specs=[pl.BlockSpec((bb,), lambda i: (i,)),
                 pl.BlockSpec((bb,), lambda i: (i,))],
      out_shape=[jax.ShapeDtypeStruct((B2,), jnp.float32),
                 jax.ShapeDtypeStruct((B2,), jnp.float32)],
  )(gu, gi, mu, mi, W1, b1, W2, b2, W3, b3, Wo, bo)
  return oe, oo


def kernel(user_indices, item_indices, user_emb_gmf, item_emb_gmf,
           user_emb_mlp, item_emb_mlp, W1, b1, W2, b2, W3, b3, Wo, bo):
  packed = _tc_pack_transpose4(
      (user_emb_gmf.T, item_emb_gmf.T, user_emb_mlp.T, item_emb_mlp.T))
  remap = lambda i: jnp.where(i < _TH, 2 * i, 2 * (i - _TH) + 1)
  views = [p.reshape(2 * _TH, 64) for p in packed]
  gu, gi, mu, mi = _sc_gather4(remap(user_indices), remap(item_indices),
                               *views)
  B, D = gu.shape
  pack = lambda a: a.reshape(B // 2, 2 * D)
  oe, oo = _tc_mlp(pack(gu), pack(gi), pack(mu), pack(mi),
                   W1, b1, W2, b2, W3, b3, Wo, bo)
  return jnp.stack((oe, oo), axis=-1).reshape(B)
